# baseline (speedup 1.0000x reference)
import functools

import jax
import jax.numpy as jnp
from jax import lax
from jax.experimental import pallas as pl
from jax.experimental.pallas import tpu as pltpu
from jax.experimental.pallas import tpu_sc as plsc

_C = 96
_CP = 128
_T = 4096
_B = 8
_NC = 2
_NS = 16


def _fill_spatial(tab_ref, n, width, fw1, jc):
    ivec = lax.broadcasted_iota(jnp.int32, (n, 1), 0)
    a_row = (ivec // width).astype(jnp.float32)
    a_col = (ivec % width).astype(jnp.float32)
    for k in range(n // jc):
        off = k * jc
        jvec = lax.broadcasted_iota(jnp.int32, (1, jc), 1) + (n + off)
        b_row = (jvec // width).astype(jnp.float32)
        b_col = (jvec % width).astype(jnp.float32)
        dr = a_row - b_row
        dc = a_col - b_col
        dist = jnp.sqrt(dr * dr + dc * dc)
        spatial = 1.0 / (dist + 1e-6)
        tab_ref[:, off:off + jc] = fw1 * spatial


def _route(xb, n, fw0, jc, tab_ref):
    norm = jnp.sqrt(jnp.sum(xb * xb, axis=0, keepdims=True))
    a_n = (xb[:, :n] / norm[:, :n]).astype(jnp.bfloat16)
    b_n = (xb[:, n:] / norm[:, n:]).astype(jnp.bfloat16)

    best_val = jnp.full((n, 1), -jnp.inf, dtype=jnp.float32)
    best_idx = jnp.zeros((n, 1), dtype=jnp.int32)
    for k in range(n // jc):
        off = k * jc
        sim = lax.dot_general(
            a_n, b_n[:, off:off + jc],
            dimension_numbers=(((0,), (0,)), ((), ())),
            preferred_element_type=jnp.float32,
        )
        combined = fw0 * sim + tab_ref[:, off:off + jc]
        loc_max = jnp.max(combined, axis=1, keepdims=True)
        jj = lax.broadcasted_iota(jnp.int32, (n, jc), 1) + off
        loc_arg = jnp.min(
            jnp.where(combined == loc_max, jj, n), axis=1, keepdims=True
        )
        upd = loc_max > best_val
        best_val = jnp.where(upd, loc_max, best_val)
        best_idx = jnp.where(upd, loc_arg, best_idx)
    return best_idx


def _pad_tokens(merged, n):
    full = jnp.concatenate(
        [merged,
         jnp.ones((1, n), jnp.float32),
         jnp.zeros((_CP - _C - 1, n), jnp.float32)], axis=0)
    return jnp.transpose(full, (1, 0))


def _k1_body(x_ref, fw_ref, xft_ref, dst_ref, tab_ref):
    @pl.when(pl.program_id(0) == 0)
    def _():
        _fill_spatial(tab_ref, _T // 2, 64, fw_ref[0, 1], 1024)

    xb = x_ref[0]
    best_idx = _route(xb, _T // 2, fw_ref[0, 0], 1024, tab_ref)
    dst_ref[0] = jnp.transpose(best_idx, (1, 0))
    xft_ref[0] = _pad_tokens(xb, _T)


def _k3_body(m1s_ref, fw_ref, dst_ref, m1m_ref, tab_ref):
    @pl.when(pl.program_id(0) == 0)
    def _():
        _fill_spatial(tab_ref, _T // 4, 45, fw_ref[1, 1], 1024)

    mt = jnp.transpose(m1s_ref[0], (1, 0))
    merged = mt[:_C] / mt[_C:_C + 1]
    best_idx = _route(merged, _T // 4, fw_ref[1, 0], 1024, tab_ref)
    dst_ref[0] = jnp.transpose(best_idx, (1, 0))
    m1m_ref[0] = _pad_tokens(merged, _T // 2)


def _k5_body(m2s_ref, w_ref, b_ref, out_ref):
    mt = jnp.transpose(m2s_ref[0], (1, 0))
    merged = mt[:_C] / mt[_C:_C + 1]
    out_ref[0] = lax.dot_general(
        w_ref[...], merged,
        dimension_numbers=(((1,), (0,)), ((), ())),
        preferred_element_type=jnp.float32,
        precision=lax.Precision.HIGHEST,
    ) + b_ref[...]


def _make_sc_merge(n):
    rpt = n // _NS
    mesh = plsc.VectorSubcoreMesh(core_axis_name="c", subcore_axis_name="s",
                                  num_cores=_NC, num_subcores=_NS)

    @functools.partial(
        pl.kernel,
        out_type=jax.ShapeDtypeStruct((_B, n, _CP), jnp.float32),
        mesh=mesh,
        scratch_types=[
            pltpu.MemorySpace.VMEM_SHARED((n, _CP), jnp.float32),
            pltpu.MemorySpace.VMEM((rpt, _CP), jnp.float32),
            pltpu.MemorySpace.VMEM((rpt,), jnp.int32),
        ],
    )
    def merge(xft_hbm, dst_hbm, out_hbm, acc, arows, idx):
        cid = lax.axis_index("c")
        sid = lax.axis_index("s")
        row0 = sid * rpt
        for k in range(_B // _NC):
            bb = cid * (_B // _NC) + k
            pltpu.sync_copy(xft_hbm.at[bb, pl.ds(n + row0, rpt)],
                            acc.at[pl.ds(row0, rpt)])
            plsc.subcore_barrier()
            pltpu.sync_copy(dst_hbm.at[bb, pl.ds(row0, rpt)], idx)
            pltpu.sync_copy(xft_hbm.at[bb, pl.ds(row0, rpt)], arows)
            pltpu.sync_copy(arows, acc.at[idx], add=True)
            plsc.subcore_barrier()
            pltpu.sync_copy(acc.at[pl.ds(row0, rpt)],
                            out_hbm.at[bb, pl.ds(row0, rpt)])
            plsc.subcore_barrier()

    return merge


@jax.jit
def kernel(x, W_conv, b_conv, w_fuse1, w_fuse2):
    B, C, H, W = x.shape
    T = H * W
    xc = x.reshape(B, C, T)

    def fw(w):
        w = jnp.clip(w, 0.0, 6.0)
        return w / (jnp.sum(w) + 1e-8)

    fws = jnp.stack([fw(w_fuse1), fw(w_fuse2)]).astype(jnp.float32)

    xft, dst1 = pl.pallas_call(
        _k1_body,
        grid=(B,),
        in_specs=[
            pl.BlockSpec((1, C, T), lambda b: (b, 0, 0)),
            pl.BlockSpec(memory_space=pltpu.SMEM),
        ],
        out_specs=[
            pl.BlockSpec((1, T, _CP), lambda b: (b, 0, 0)),
            pl.BlockSpec((1, 1, T // 2), lambda b: (b, 0, 0)),
        ],
        out_shape=[
            jax.ShapeDtypeStruct((B, T, _CP), jnp.float32),
            jax.ShapeDtypeStruct((B, 1, T // 2), jnp.int32),
        ],
        scratch_shapes=[pltpu.VMEM((_T // 2, _T // 2), jnp.float32)],
        compiler_params=pltpu.CompilerParams(
            dimension_semantics=("arbitrary",),
        ),
    )(xc, fws)

    m1s = _make_sc_merge(T // 2)(xft, dst1.reshape(B, T // 2))

    dst2, m1m = pl.pallas_call(
        _k3_body,
        grid=(B,),
        in_specs=[
            pl.BlockSpec((1, T // 2, _CP), lambda b: (b, 0, 0)),
            pl.BlockSpec(memory_space=pltpu.SMEM),
        ],
        out_specs=[
            pl.BlockSpec((1, 1, T // 4), lambda b: (b, 0, 0)),
            pl.BlockSpec((1, T // 2, _CP), lambda b: (b, 0, 0)),
        ],
        out_shape=[
            jax.ShapeDtypeStruct((B, 1, T // 4), jnp.int32),
            jax.ShapeDtypeStruct((B, T // 2, _CP), jnp.float32),
        ],
        scratch_shapes=[pltpu.VMEM((_T // 4, _T // 4), jnp.float32)],
        compiler_params=pltpu.CompilerParams(
            dimension_semantics=("arbitrary",),
        ),
    )(m1s, fws)

    m2s = _make_sc_merge(T // 4)(m1m, dst2.reshape(B, T // 4))

    out = pl.pallas_call(
        _k5_body,
        grid=(B,),
        in_specs=[
            pl.BlockSpec((1, T // 4, _CP), lambda b: (b, 0, 0)),
            pl.BlockSpec((C, C), lambda b: (0, 0)),
            pl.BlockSpec((C, 1), lambda b: (0, 0)),
        ],
        out_specs=pl.BlockSpec((1, C, T // 4), lambda b: (b, 0, 0)),
        out_shape=jax.ShapeDtypeStruct((B, C, T // 4), jnp.float32),
        compiler_params=pltpu.CompilerParams(
            dimension_semantics=("parallel",),
        ),
    )(m2s, W_conv, b_conv.reshape(C, 1))
    return out.reshape(B, C, H // 2, W // 2)


# speedup vs baseline: 1.0894x; 1.0894x over previous
import functools

import jax
import jax.numpy as jnp
from jax import lax
from jax.experimental import pallas as pl
from jax.experimental.pallas import tpu as pltpu
from jax.experimental.pallas import tpu_sc as plsc

_C = 96
_CP = 128
_T = 4096
_B = 8
_NC = 2
_NS = 16


def _fill_spatial(tab_ref, n, width, fw1, jc):
    ivec = lax.broadcasted_iota(jnp.int32, (n, 1), 0)
    a_row = (ivec // width).astype(jnp.float32)
    a_col = (ivec % width).astype(jnp.float32)
    for k in range(n // jc):
        off = k * jc
        jvec = lax.broadcasted_iota(jnp.int32, (1, jc), 1) + (n + off)
        b_row = (jvec // width).astype(jnp.float32)
        b_col = (jvec % width).astype(jnp.float32)
        dr = a_row - b_row
        dc = a_col - b_col
        dist = jnp.sqrt(dr * dr + dc * dc)
        spatial = 1.0 / (dist + 1e-6)
        tab_ref[:, off:off + jc] = fw1 * spatial


def _route(xb, n, fw0, jc, tab_ref):
    norm = jnp.sqrt(jnp.sum(xb * xb, axis=0, keepdims=True))
    a_n = (xb[:, :n] / norm[:, :n]).astype(jnp.bfloat16)
    b_n = (xb[:, n:] / norm[:, n:]).astype(jnp.bfloat16)

    best_val = jnp.full((n, 1), -jnp.inf, dtype=jnp.float32)
    best_idx = jnp.zeros((n, 1), dtype=jnp.int32)
    for k in range(n // jc):
        off = k * jc
        sim = lax.dot_general(
            a_n, b_n[:, off:off + jc],
            dimension_numbers=(((0,), (0,)), ((), ())),
            preferred_element_type=jnp.float32,
        )
        combined = fw0 * sim + tab_ref[:, off:off + jc]
        loc_max = jnp.max(combined, axis=1, keepdims=True)
        jj = lax.broadcasted_iota(jnp.int32, (n, jc), 1) + off
        loc_arg = jnp.min(
            jnp.where(combined == loc_max, jj, n), axis=1, keepdims=True
        )
        upd = loc_max > best_val
        best_val = jnp.where(upd, loc_max, best_val)
        best_idx = jnp.where(upd, loc_arg, best_idx)
    return best_idx


def _pad_tokens(merged, n):
    full = jnp.concatenate(
        [merged,
         jnp.ones((1, n), jnp.float32),
         jnp.zeros((_CP - _C - 1, n), jnp.float32)], axis=0)
    return jnp.transpose(full, (1, 0))


def _k1_body(x_ref, fw_ref, xft_ref, dst_ref, tab_ref):
    @pl.when(pl.program_id(0) == 0)
    def _():
        _fill_spatial(tab_ref, _T // 2, 64, fw_ref[0, 1], 1024)

    xb = x_ref[0]
    best_idx = _route(xb, _T // 2, fw_ref[0, 0], 1024, tab_ref)
    dst_ref[0] = jnp.transpose(best_idx, (1, 0))
    xft_ref[0] = _pad_tokens(xb, _T)


def _k3_body(m1s_ref, fw_ref, dst_ref, m1m_ref, tab_ref):
    @pl.when(pl.program_id(0) == 0)
    def _():
        _fill_spatial(tab_ref, _T // 4, 45, fw_ref[1, 1], 1024)

    mt = jnp.transpose(m1s_ref[0], (1, 0))
    merged = mt[:_C] / mt[_C:_C + 1]
    best_idx = _route(merged, _T // 4, fw_ref[1, 0], 1024, tab_ref)
    dst_ref[0] = jnp.transpose(best_idx, (1, 0))
    m1m_ref[0] = _pad_tokens(merged, _T // 2)


def _k5_body(m2s_ref, w_ref, b_ref, out_ref):
    mt = jnp.transpose(m2s_ref[0], (1, 0))
    merged = mt[:_C] / mt[_C:_C + 1]
    out_ref[0] = lax.dot_general(
        w_ref[...], merged,
        dimension_numbers=(((1,), (0,)), ((), ())),
        preferred_element_type=jnp.float32,
        precision=lax.Precision.HIGHEST,
    ) + b_ref[...]


def _make_sc_merge(n):
    rpt = n // _NS
    mesh = plsc.VectorSubcoreMesh(core_axis_name="c", subcore_axis_name="s",
                                  num_cores=_NC, num_subcores=_NS)

    @functools.partial(
        pl.kernel,
        out_type=jax.ShapeDtypeStruct((_B, n, _CP), jnp.float32),
        mesh=mesh,
        scratch_types=[
            pltpu.MemorySpace.VMEM_SHARED((n, _CP), jnp.float32),
            pltpu.MemorySpace.VMEM((rpt, _CP), jnp.float32),
            pltpu.MemorySpace.VMEM((rpt,), jnp.int32),
            pltpu.SemaphoreType.DMA,
        ],
    )
    def merge(xft_hbm, dst_hbm, out_hbm, acc, arows, idx, sem):
        cid = lax.axis_index("c")
        sid = lax.axis_index("s")
        row0 = sid * rpt
        for k in range(_B // _NC):
            bb = cid * (_B // _NC) + k
            d1 = pltpu.async_copy(xft_hbm.at[bb, pl.ds(n + row0, rpt)],
                                  acc.at[pl.ds(row0, rpt)], sem)
            d2 = pltpu.async_copy(dst_hbm.at[bb, pl.ds(row0, rpt)], idx, sem)
            d3 = pltpu.async_copy(xft_hbm.at[bb, pl.ds(row0, rpt)], arows,
                                  sem)
            d1.wait()
            d2.wait()
            d3.wait()
            plsc.subcore_barrier()
            pltpu.sync_copy(arows, acc.at[idx], add=True)
            plsc.subcore_barrier()
            pltpu.sync_copy(acc.at[pl.ds(row0, rpt)],
                            out_hbm.at[bb, pl.ds(row0, rpt)])

    return merge


@jax.jit
def kernel(x, W_conv, b_conv, w_fuse1, w_fuse2):
    B, C, H, W = x.shape
    T = H * W
    xc = x.reshape(B, C, T)

    def fw(w):
        w = jnp.clip(w, 0.0, 6.0)
        return w / (jnp.sum(w) + 1e-8)

    fws = jnp.stack([fw(w_fuse1), fw(w_fuse2)]).astype(jnp.float32)

    xft, dst1 = pl.pallas_call(
        _k1_body,
        grid=(B,),
        in_specs=[
            pl.BlockSpec((1, C, T), lambda b: (b, 0, 0)),
            pl.BlockSpec(memory_space=pltpu.SMEM),
        ],
        out_specs=[
            pl.BlockSpec((1, T, _CP), lambda b: (b, 0, 0)),
            pl.BlockSpec((1, 1, T // 2), lambda b: (b, 0, 0)),
        ],
        out_shape=[
            jax.ShapeDtypeStruct((B, T, _CP), jnp.float32),
            jax.ShapeDtypeStruct((B, 1, T // 2), jnp.int32),
        ],
        scratch_shapes=[pltpu.VMEM((_T // 2, _T // 2), jnp.float32)],
        compiler_params=pltpu.CompilerParams(
            dimension_semantics=("arbitrary",),
        ),
    )(xc, fws)

    m1s = _make_sc_merge(T // 2)(xft, dst1.reshape(B, T // 2))

    dst2, m1m = pl.pallas_call(
        _k3_body,
        grid=(B,),
        in_specs=[
            pl.BlockSpec((1, T // 2, _CP), lambda b: (b, 0, 0)),
            pl.BlockSpec(memory_space=pltpu.SMEM),
        ],
        out_specs=[
            pl.BlockSpec((1, 1, T // 4), lambda b: (b, 0, 0)),
            pl.BlockSpec((1, T // 2, _CP), lambda b: (b, 0, 0)),
        ],
        out_shape=[
            jax.ShapeDtypeStruct((B, 1, T // 4), jnp.int32),
            jax.ShapeDtypeStruct((B, T // 2, _CP), jnp.float32),
        ],
        scratch_shapes=[pltpu.VMEM((_T // 4, _T // 4), jnp.float32)],
        compiler_params=pltpu.CompilerParams(
            dimension_semantics=("arbitrary",),
        ),
    )(m1s, fws)

    m2s = _make_sc_merge(T // 4)(m1m, dst2.reshape(B, T // 4))

    out = pl.pallas_call(
        _k5_body,
        grid=(B,),
        in_specs=[
            pl.BlockSpec((1, T // 4, _CP), lambda b: (b, 0, 0)),
            pl.BlockSpec((C, C), lambda b: (0, 0)),
            pl.BlockSpec((C, 1), lambda b: (0, 0)),
        ],
        out_specs=pl.BlockSpec((1, C, T // 4), lambda b: (b, 0, 0)),
        out_shape=jax.ShapeDtypeStruct((B, C, T // 4), jnp.float32),
        compiler_params=pltpu.CompilerParams(
            dimension_semantics=("parallel",),
        ),
    )(m2s, W_conv, b_conv.reshape(C, 1))
    return out.reshape(B, C, H // 2, W // 2)


# speedup vs baseline: 1.1376x; 1.0443x over previous
import functools

import jax
import jax.numpy as jnp
from jax import lax
from jax.experimental import pallas as pl
from jax.experimental.pallas import tpu as pltpu
from jax.experimental.pallas import tpu_sc as plsc

_C = 96
_CP = 128
_T = 4096
_B = 8
_NC = 2
_NS = 16


def _fill_spatial(tab_ref, n, width, fw1, jc):
    ivec = lax.broadcasted_iota(jnp.int32, (n, 1), 0)
    a_row = (ivec // width).astype(jnp.float32)
    a_col = (ivec % width).astype(jnp.float32)
    for k in range(n // jc):
        off = k * jc
        jvec = lax.broadcasted_iota(jnp.int32, (1, jc), 1) + (n + off)
        b_row = (jvec // width).astype(jnp.float32)
        b_col = (jvec % width).astype(jnp.float32)
        dr = a_row - b_row
        dc = a_col - b_col
        dist = jnp.sqrt(dr * dr + dc * dc)
        spatial = 1.0 / (dist + 1e-6)
        tab_ref[:, off:off + jc] = fw1 * spatial


def _route(xb, n, fw0, jc, tab_ref):
    norm = jnp.sqrt(jnp.sum(xb * xb, axis=0, keepdims=True))
    a_n = (xb[:, :n] / norm[:, :n]).astype(jnp.bfloat16)
    b_n = (xb[:, n:] / norm[:, n:]).astype(jnp.bfloat16)

    best_val = jnp.full((n, 1), -jnp.inf, dtype=jnp.float32)
    best_idx = jnp.zeros((n, 1), dtype=jnp.int32)
    for k in range(n // jc):
        off = k * jc
        sim = lax.dot_general(
            a_n, b_n[:, off:off + jc],
            dimension_numbers=(((0,), (0,)), ((), ())),
            preferred_element_type=jnp.float32,
        )
        combined = fw0 * sim + tab_ref[:, off:off + jc]
        loc_max = jnp.max(combined, axis=1, keepdims=True)
        jj = lax.broadcasted_iota(jnp.int32, (n, jc), 1) + off
        loc_arg = jnp.min(
            jnp.where(combined == loc_max, jj, n), axis=1, keepdims=True
        )
        upd = loc_max > best_val
        best_val = jnp.where(upd, loc_max, best_val)
        best_idx = jnp.where(upd, loc_arg, best_idx)
    return best_idx


def _pad_tokens(merged, n):
    full = jnp.concatenate(
        [merged,
         jnp.ones((1, n), jnp.float32),
         jnp.zeros((_CP - _C - 1, n), jnp.float32)], axis=0)
    return jnp.transpose(full, (1, 0))


def _k1_body(x_ref, fw_ref, xft_ref, dst_ref, tab_ref):
    @pl.when(pl.program_id(0) == 0)
    def _():
        _fill_spatial(tab_ref, _T // 2, 64, fw_ref[0, 1], 1024)

    xb = x_ref[0]
    best_idx = _route(xb, _T // 2, fw_ref[0, 0], 1024, tab_ref)
    dst_ref[0] = jnp.transpose(best_idx, (1, 0))
    xft_ref[0] = _pad_tokens(xb, _T)


def _k3_body(m1s_ref, fw_ref, dst_ref, m1m_ref, tab_ref):
    @pl.when(pl.program_id(0) == 0)
    def _():
        _fill_spatial(tab_ref, _T // 4, 45, fw_ref[1, 1], 1024)

    mt = jnp.transpose(m1s_ref[0], (1, 0))
    merged = mt[:_C] / mt[_C:_C + 1]
    best_idx = _route(merged, _T // 4, fw_ref[1, 0], 1024, tab_ref)
    dst_ref[0] = jnp.transpose(best_idx, (1, 0))
    m1m_ref[0] = _pad_tokens(merged, _T // 2)


def _k5_body(m2s_ref, w_ref, b_ref, out_ref):
    mt = jnp.transpose(m2s_ref[0], (1, 0))
    merged = mt[:_C] / mt[_C:_C + 1]
    out_ref[0] = lax.dot_general(
        w_ref[...], merged,
        dimension_numbers=(((1,), (0,)), ((), ())),
        preferred_element_type=jnp.float32,
        precision=lax.Precision.HIGHEST,
    ) + b_ref[...]


def _make_sc_merge(n):
    rpt = n // _NS
    mesh = plsc.VectorSubcoreMesh(core_axis_name="c", subcore_axis_name="s",
                                  num_cores=_NC, num_subcores=_NS)

    @functools.partial(
        pl.kernel,
        out_type=jax.ShapeDtypeStruct((_B, n, _CP), jnp.float32),
        mesh=mesh,
        scratch_types=[
            pltpu.MemorySpace.VMEM_SHARED((2, n, _CP), jnp.float32),
            pltpu.MemorySpace.VMEM((2, rpt, _CP), jnp.float32),
            pltpu.MemorySpace.VMEM((2, rpt), jnp.int32),
            pltpu.SemaphoreType.DMA,
        ],
    )
    def merge(xft_hbm, dst_hbm, out_hbm, acc, arows, idx, sem):
        cid = lax.axis_index("c")
        sid = lax.axis_index("s")
        row0 = sid * rpt
        nb = _B // _NC

        def issue_reads(k, s):
            bb = cid * nb + k
            d1 = pltpu.async_copy(xft_hbm.at[bb, pl.ds(n + row0, rpt)],
                                  acc.at[s, pl.ds(row0, rpt)], sem)
            d2 = pltpu.async_copy(dst_hbm.at[bb, pl.ds(row0, rpt)],
                                  idx.at[s], sem)
            d3 = pltpu.async_copy(xft_hbm.at[bb, pl.ds(row0, rpt)],
                                  arows.at[s], sem)
            return d1, d2, d3

        pend = issue_reads(0, 0)
        for k in range(nb):
            s = k % 2
            bb = cid * nb + k
            for d in pend:
                d.wait()
            plsc.subcore_barrier()
            if k + 1 < nb:
                pend = issue_reads(k + 1, (k + 1) % 2)
            pltpu.sync_copy(arows.at[s], acc.at[s].at[idx.at[s]], add=True)
            plsc.subcore_barrier()
            pltpu.sync_copy(acc.at[s, pl.ds(row0, rpt)],
                            out_hbm.at[bb, pl.ds(row0, rpt)])

    return merge


@jax.jit
def kernel(x, W_conv, b_conv, w_fuse1, w_fuse2):
    B, C, H, W = x.shape
    T = H * W
    xc = x.reshape(B, C, T)

    def fw(w):
        w = jnp.clip(w, 0.0, 6.0)
        return w / (jnp.sum(w) + 1e-8)

    fws = jnp.stack([fw(w_fuse1), fw(w_fuse2)]).astype(jnp.float32)

    xft, dst1 = pl.pallas_call(
        _k1_body,
        grid=(B,),
        in_specs=[
            pl.BlockSpec((1, C, T), lambda b: (b, 0, 0)),
            pl.BlockSpec(memory_space=pltpu.SMEM),
        ],
        out_specs=[
            pl.BlockSpec((1, T, _CP), lambda b: (b, 0, 0)),
            pl.BlockSpec((1, 1, T // 2), lambda b: (b, 0, 0)),
        ],
        out_shape=[
            jax.ShapeDtypeStruct((B, T, _CP), jnp.float32),
            jax.ShapeDtypeStruct((B, 1, T // 2), jnp.int32),
        ],
        scratch_shapes=[pltpu.VMEM((_T // 2, _T // 2), jnp.float32)],
        compiler_params=pltpu.CompilerParams(
            dimension_semantics=("arbitrary",),
        ),
    )(xc, fws)

    m1s = _make_sc_merge(T // 2)(xft, dst1.reshape(B, T // 2))

    dst2, m1m = pl.pallas_call(
        _k3_body,
        grid=(B,),
        in_specs=[
            pl.BlockSpec((1, T // 2, _CP), lambda b: (b, 0, 0)),
            pl.BlockSpec(memory_space=pltpu.SMEM),
        ],
        out_specs=[
            pl.BlockSpec((1, 1, T // 4), lambda b: (b, 0, 0)),
            pl.BlockSpec((1, T // 2, _CP), lambda b: (b, 0, 0)),
        ],
        out_shape=[
            jax.ShapeDtypeStruct((B, 1, T // 4), jnp.int32),
            jax.ShapeDtypeStruct((B, T // 2, _CP), jnp.float32),
        ],
        scratch_shapes=[pltpu.VMEM((_T // 4, _T // 4), jnp.float32)],
        compiler_params=pltpu.CompilerParams(
            dimension_semantics=("arbitrary",),
        ),
    )(m1s, fws)

    m2s = _make_sc_merge(T // 4)(m1m, dst2.reshape(B, T // 4))

    out = pl.pallas_call(
        _k5_body,
        grid=(B,),
        in_specs=[
            pl.BlockSpec((1, T // 4, _CP), lambda b: (b, 0, 0)),
            pl.BlockSpec((C, C), lambda b: (0, 0)),
            pl.BlockSpec((C, 1), lambda b: (0, 0)),
        ],
        out_specs=pl.BlockSpec((1, C, T // 4), lambda b: (b, 0, 0)),
        out_shape=jax.ShapeDtypeStruct((B, C, T // 4), jnp.float32),
        compiler_params=pltpu.CompilerParams(
            dimension_semantics=("parallel",),
        ),
    )(m2s, W_conv, b_conv.reshape(C, 1))
    return out.reshape(B, C, H // 2, W // 2)
